# baseline (speedup 1.0000x reference)
import jax
import jax.numpy as jnp
from jax import lax
from jax.experimental import pallas as pl
from jax.experimental.pallas import tpu as pltpu
from jax.experimental.pallas import tpu_sc as plsc

N = 10000
D = 256
H = 128
E = 160000
NC, NS = 2, 16

_EC = E // (NC * NS)
_ROWS_A = 640
_NPAD = _ROWS_A * NS


def _deg_body(dst_hbm, degs0_hbm, degs1_hbm, dstv, ones_v, zb, deg_sh):
    c = lax.axis_index("c")
    s = lax.axis_index("s")

    def fill_ones(i, _):
        ones_v[pl.ds(i * 16, 16)] = jnp.ones((16,), jnp.float32)
        return 0

    lax.fori_loop(0, _EC // 16, fill_ones, 0)
    ones_v[pl.ds(_EC - 16, 16)] = jnp.ones((16,), jnp.float32)

    def fill_zero(i, _):
        zb[pl.ds(i * 16, 16)] = jnp.zeros((16,), jnp.float32)
        return 0

    lax.fori_loop(0, _ROWS_A // 16, fill_zero, 0)
    pltpu.sync_copy(zb, deg_sh.at[pl.ds(s * _ROWS_A, _ROWS_A)])
    plsc.subcore_barrier()
    off = c * (E // 2) + s * _EC
    pltpu.sync_copy(dst_hbm.at[pl.ds(off, _EC)], dstv)
    pltpu.sync_copy(ones_v, deg_sh.at[dstv], add=True)
    plsc.subcore_barrier()
    pltpu.sync_copy(deg_sh.at[pl.ds(s * _ROWS_A, _ROWS_A)], zb)

    @pl.when(c == 0)
    def _():
        pltpu.sync_copy(zb, degs0_hbm.at[pl.ds(s * _ROWS_A, _ROWS_A)])

    @pl.when(c == 1)
    def _():
        pltpu.sync_copy(zb, degs1_hbm.at[pl.ds(s * _ROWS_A, _ROWS_A)])


_deg_call = pl.kernel(
    _deg_body,
    out_type=[
        jax.ShapeDtypeStruct((_NPAD,), jnp.float32),
        jax.ShapeDtypeStruct((_NPAD,), jnp.float32),
    ],
    mesh=plsc.VectorSubcoreMesh(core_axis_name="c", subcore_axis_name="s"),
    scratch_types=[
        pltpu.VMEM((_EC,), jnp.int32),
        pltpu.VMEM((_EC,), jnp.float32),
        pltpu.VMEM((_ROWS_A,), jnp.float32),
        pltpu.VMEM_SHARED((_NPAD,), jnp.float32),
    ],
)

_RB = 1000


def _mm_body(x_ref, w_ref, d0_ref, d1_ref, yl_ref, yr_ref, dis_ref):
    deg = d0_ref[...] + d1_ref[...] + 1.0
    dis = lax.rsqrt(deg)
    xw = jnp.dot(x_ref[...], w_ref[...], preferred_element_type=jnp.float32)
    y = xw * dis
    yl_ref[...] = y[:, :H]
    yr_ref[...] = y[:, H:]
    dis_ref[...] = dis


_mm_call = pl.pallas_call(
    _mm_body,
    grid=(N // _RB,),
    in_specs=[
        pl.BlockSpec((_RB, D), lambda i: (i, 0)),
        pl.BlockSpec((D, D), lambda i: (0, 0)),
        pl.BlockSpec((_RB, 1), lambda i: (i, 0)),
        pl.BlockSpec((_RB, 1), lambda i: (i, 0)),
    ],
    out_specs=[
        pl.BlockSpec((_RB, H), lambda i: (i, 0)),
        pl.BlockSpec((_RB, H), lambda i: (i, 0)),
        pl.BlockSpec((_RB, 1), lambda i: (i, 0)),
    ],
    out_shape=[
        jax.ShapeDtypeStruct((N, H), jnp.float32),
        jax.ShapeDtypeStruct((N, H), jnp.float32),
        jax.ShapeDtypeStruct((N, 1), jnp.float32),
    ],
)

_BE = 112
_ET = E // NS
_NB = 90
_ETP = _NB * _BE
_NP = _NB // 2
_RT = 624


def _edge_body(yl_hbm, yr_hbm, src_hbm, dst_hbm, accl_hbm, accr_hbm,
               idx_s, idx_d, rows_a, rows_b, acc_sh, sem_a, sem_b, sem_sa,
               sem_sb, sem_p):
    c = lax.axis_index("c")
    s = lax.axis_index("s")
    e0 = s * _ETP
    pltpu.sync_copy(src_hbm.at[pl.ds(e0, _ETP)], idx_s)

    def pre_start(b, _):
        pltpu.async_copy(dst_hbm.at[pl.ds(e0 + b * _BE, _BE)], idx_d.at[b],
                         sem_p)
        return 0

    lax.fori_loop(0, _NB, pre_start, 0)

    def fill_zero(i, _):
        rows_a[i // 8, pl.ds((i % 8) * 16, 16)] = jnp.zeros((16,), jnp.float32)
        return 0

    lax.fori_loop(0, 16 * (H // 16), fill_zero, 0)
    r0 = s * _RT
    nzb = jnp.where(s < 15, _RT // 16, 640 // 16)

    def zero_dma(j, _):
        pltpu.sync_copy(rows_a.at[pl.ds(0, 16)], acc_sh.at[pl.ds(r0 + 16 * j, 16)])
        return 0

    lax.fori_loop(0, nzb, zero_dma, 0)

    def pre_drain(b, _):
        pltpu.make_async_copy(dst_hbm.at[pl.ds(e0 + b * _BE, _BE)],
                              idx_d.at[b], sem_p).wait()
        return 0

    lax.fori_loop(0, _NB, pre_drain, 0)
    plsc.subcore_barrier()

    def start(b, rows, sem):
        @pl.when(c == 0)
        def _():
            pltpu.async_copy(yl_hbm.at[idx_s.at[pl.ds(b * _BE, _BE)]], rows, sem)

        @pl.when(c == 1)
        def _():
            pltpu.async_copy(yr_hbm.at[idx_s.at[pl.ds(b * _BE, _BE)]], rows, sem)

    def wait(b, rows, sem):
        @pl.when(c == 0)
        def _():
            pltpu.make_async_copy(yl_hbm.at[idx_s.at[pl.ds(b * _BE, _BE)]],
                                  rows, sem).wait()

        @pl.when(c == 1)
        def _():
            pltpu.make_async_copy(yr_hbm.at[idx_s.at[pl.ds(b * _BE, _BE)]],
                                  rows, sem).wait()

    def scat_start(b, rows, sem):
        pltpu.async_copy(rows, acc_sh.at[idx_d.at[b]], sem, add=True)

    def scat_wait(b, rows, sem):
        pltpu.make_async_copy(rows, acc_sh.at[idx_d.at[b]], sem).wait()

    start(0, rows_a, sem_a)
    start(1, rows_b, sem_b)

    def pair(g, _):
        b0 = 2 * g
        wait(b0, rows_a, sem_a)
        scat_start(b0, rows_a, sem_sa)
        wait(b0 + 1, rows_b, sem_b)
        scat_start(b0 + 1, rows_b, sem_sb)
        scat_wait(b0, rows_a, sem_sa)

        @pl.when(g < _NP - 1)
        def _():
            start(b0 + 2, rows_a, sem_a)

        scat_wait(b0 + 1, rows_b, sem_sb)

        @pl.when(g < _NP - 1)
        def _():
            start(b0 + 3, rows_b, sem_b)

        return 0

    lax.fori_loop(0, _NP, pair, 0)
    plsc.subcore_barrier()

    @pl.when((c == 0) & (s < 15))
    def _():
        pltpu.sync_copy(acc_sh.at[pl.ds(r0, _RT)], accl_hbm.at[pl.ds(r0, _RT)])

    @pl.when((c == 0) & (s == 15))
    def _():
        pltpu.sync_copy(acc_sh.at[pl.ds(15 * _RT, 640)],
                        accl_hbm.at[pl.ds(15 * _RT, 640)])

    @pl.when((c == 1) & (s < 15))
    def _():
        pltpu.sync_copy(acc_sh.at[pl.ds(r0, _RT)], accr_hbm.at[pl.ds(r0, _RT)])

    @pl.when((c == 1) & (s == 15))
    def _():
        pltpu.sync_copy(acc_sh.at[pl.ds(15 * _RT, 640)],
                        accr_hbm.at[pl.ds(15 * _RT, 640)])


_edge_call = pl.kernel(
    _edge_body,
    out_type=[
        jax.ShapeDtypeStruct((N, H), jnp.float32),
        jax.ShapeDtypeStruct((N, H), jnp.float32),
    ],
    mesh=plsc.VectorSubcoreMesh(core_axis_name="c", subcore_axis_name="s"),
    scratch_types=[
        pltpu.VMEM((_ETP,), jnp.int32),
        pltpu.VMEM((_NB, _BE), jnp.int32),
        pltpu.VMEM((_BE, H), jnp.float32),
        pltpu.VMEM((_BE, H), jnp.float32),
        pltpu.VMEM_SHARED((N, H), jnp.float32),
        pltpu.SemaphoreType.DMA,
        pltpu.SemaphoreType.DMA,
        pltpu.SemaphoreType.DMA,
        pltpu.SemaphoreType.DMA,
        pltpu.SemaphoreType.DMA,
    ],
)


def _fin_body(al_ref, ar_ref, yl_ref, yr_ref, dis_ref, b_ref, out_ref):
    d = dis_ref[...]
    left = (al_ref[...] + yl_ref[...]) * d
    right = (ar_ref[...] + yr_ref[...]) * d
    out_ref[...] = jnp.concatenate([left, right], axis=1) + b_ref[...]


_fin_call = pl.pallas_call(
    _fin_body,
    grid=(N // _RB,),
    in_specs=[
        pl.BlockSpec((_RB, H), lambda i: (i, 0)),
        pl.BlockSpec((_RB, H), lambda i: (i, 0)),
        pl.BlockSpec((_RB, H), lambda i: (i, 0)),
        pl.BlockSpec((_RB, H), lambda i: (i, 0)),
        pl.BlockSpec((_RB, 1), lambda i: (i, 0)),
        pl.BlockSpec((1, D), lambda i: (0, 0)),
    ],
    out_specs=pl.BlockSpec((_RB, D), lambda i: (i, 0)),
    out_shape=jax.ShapeDtypeStruct((N, D), jnp.float32),
)


def kernel(x, edge_index, W, b):
    assert x.shape == (N, D) and W.shape == (D, D) and edge_index.shape == (2, E)
    src = edge_index[0].astype(jnp.int32)
    dst = edge_index[1].astype(jnp.int32)
    degs0, degs1 = _deg_call(dst)
    d0 = degs0[:N].reshape(N, 1)
    d1 = degs1[:N].reshape(N, 1)
    yl, yr, dis = _mm_call(x, W, d0, d1)
    pad = ((0, 0), (0, _ETP - _ET))
    srcp = jnp.pad(src.reshape(NS, _ET), pad, constant_values=N).reshape(NS * _ETP)
    dstp = jnp.pad(dst.reshape(NS, _ET), pad).reshape(NS * _ETP)
    ypad = ((0, 8), (0, 0))
    accl, accr = _edge_call(jnp.pad(yl, ypad), jnp.pad(yr, ypad), srcp, dstp)
    out = _fin_call(accl, accr, yl, yr, dis, b.reshape(1, D))
    return out


# speedup vs baseline: 1.4744x; 1.4744x over previous
import jax
import jax.numpy as jnp
from jax import lax
from jax.experimental import pallas as pl
from jax.experimental.pallas import tpu as pltpu
from jax.experimental.pallas import tpu_sc as plsc

N = 10000
D = 256
H = 128
E = 160000
NC, NS = 2, 16

_EC = E // (NC * NS)
_ROWS_A = 640
_NPAD = _ROWS_A * NS


def _deg_body(dst_hbm, degs0_hbm, degs1_hbm, dstv, ones_v, zb, deg_sh):
    c = lax.axis_index("c")
    s = lax.axis_index("s")

    def fill_ones(i, _):
        ones_v[pl.ds(i * 16, 16)] = jnp.ones((16,), jnp.float32)
        return 0

    lax.fori_loop(0, _EC // 16, fill_ones, 0)
    ones_v[pl.ds(_EC - 16, 16)] = jnp.ones((16,), jnp.float32)

    def fill_zero(i, _):
        zb[pl.ds(i * 16, 16)] = jnp.zeros((16,), jnp.float32)
        return 0

    lax.fori_loop(0, _ROWS_A // 16, fill_zero, 0)
    pltpu.sync_copy(zb, deg_sh.at[pl.ds(s * _ROWS_A, _ROWS_A)])
    plsc.subcore_barrier()
    off = c * (E // 2) + s * _EC
    pltpu.sync_copy(dst_hbm.at[pl.ds(off, _EC)], dstv)
    pltpu.sync_copy(ones_v, deg_sh.at[dstv], add=True)
    plsc.subcore_barrier()
    pltpu.sync_copy(deg_sh.at[pl.ds(s * _ROWS_A, _ROWS_A)], zb)

    @pl.when(c == 0)
    def _():
        pltpu.sync_copy(zb, degs0_hbm.at[pl.ds(s * _ROWS_A, _ROWS_A)])

    @pl.when(c == 1)
    def _():
        pltpu.sync_copy(zb, degs1_hbm.at[pl.ds(s * _ROWS_A, _ROWS_A)])


_deg_call = pl.kernel(
    _deg_body,
    out_type=[
        jax.ShapeDtypeStruct((_NPAD,), jnp.float32),
        jax.ShapeDtypeStruct((_NPAD,), jnp.float32),
    ],
    mesh=plsc.VectorSubcoreMesh(core_axis_name="c", subcore_axis_name="s"),
    scratch_types=[
        pltpu.VMEM((_EC,), jnp.int32),
        pltpu.VMEM((_EC,), jnp.float32),
        pltpu.VMEM((_ROWS_A,), jnp.float32),
        pltpu.VMEM_SHARED((_NPAD,), jnp.float32),
    ],
)

_RB = 1000


def _mm_body(x_ref, w_ref, d0_ref, d1_ref, yl_ref, yr_ref, dis_ref):
    deg = d0_ref[...] + d1_ref[...] + 1.0
    dis = lax.rsqrt(deg)
    xw = jnp.dot(x_ref[...], w_ref[...], preferred_element_type=jnp.float32)
    y = xw * dis
    yl_ref[...] = y[:, :H]
    yr_ref[...] = y[:, H:]
    dis_ref[...] = dis


_mm_call = pl.pallas_call(
    _mm_body,
    grid=(N // _RB,),
    in_specs=[
        pl.BlockSpec((_RB, D), lambda i: (i, 0)),
        pl.BlockSpec((D, D), lambda i: (0, 0)),
        pl.BlockSpec((_RB, 1), lambda i: (i, 0)),
        pl.BlockSpec((_RB, 1), lambda i: (i, 0)),
    ],
    out_specs=[
        pl.BlockSpec((_RB, H), lambda i: (i, 0)),
        pl.BlockSpec((_RB, H), lambda i: (i, 0)),
        pl.BlockSpec((_RB, 1), lambda i: (i, 0)),
    ],
    out_shape=[
        jax.ShapeDtypeStruct((N, H), jnp.float32),
        jax.ShapeDtypeStruct((N, H), jnp.float32),
        jax.ShapeDtypeStruct((N, 1), jnp.float32),
    ],
)

_BE = 104
_ET = E // NS
_NB = _ET // _BE
_TL = _ET - _NB * _BE
_NP = _NB // 2
_RT = 624


def _edge_body(yl_hbm, yr_hbm, src_hbm, dst_hbm, accl_hbm, accr_hbm,
               idx_s, idx_d, idx_t, rows_a, rows_b, acc_sh, sem_a, sem_b,
               sem_p):
    c = lax.axis_index("c")
    s = lax.axis_index("s")
    e0 = s * _ET
    pltpu.sync_copy(src_hbm.at[pl.ds(e0, _ET)], idx_s)
    pltpu.async_copy(dst_hbm.at[pl.ds(e0 + _NB * _BE, _TL)], idx_t, sem_p)

    def pre_start(b, _):
        pltpu.async_copy(dst_hbm.at[pl.ds(e0 + b * _BE, _BE)], idx_d.at[b],
                         sem_p)
        return 0

    lax.fori_loop(0, _NB, pre_start, 0)

    def fill_zero(i, _):
        rows_a[i // 8, pl.ds((i % 8) * 16, 16)] = jnp.zeros((16,), jnp.float32)
        return 0

    lax.fori_loop(0, 16 * (H // 16), fill_zero, 0)
    r0 = s * _RT
    nzb = jnp.where(s < 15, _RT // 16, 640 // 16)

    def zero_dma(j, _):
        pltpu.sync_copy(rows_a.at[pl.ds(0, 16)], acc_sh.at[pl.ds(r0 + 16 * j, 16)])
        return 0

    lax.fori_loop(0, nzb, zero_dma, 0)

    def pre_drain(b, _):
        pltpu.make_async_copy(dst_hbm.at[pl.ds(e0 + b * _BE, _BE)],
                              idx_d.at[b], sem_p).wait()
        return 0

    lax.fori_loop(0, _NB, pre_drain, 0)
    pltpu.make_async_copy(dst_hbm.at[pl.ds(e0 + _NB * _BE, _TL)], idx_t,
                          sem_p).wait()
    plsc.subcore_barrier()

    def start(b, rows, sem):
        @pl.when(c == 0)
        def _():
            pltpu.async_copy(yl_hbm.at[idx_s.at[pl.ds(b * _BE, _BE)]], rows, sem)

        @pl.when(c == 1)
        def _():
            pltpu.async_copy(yr_hbm.at[idx_s.at[pl.ds(b * _BE, _BE)]], rows, sem)

    def wait(b, rows, sem):
        @pl.when(c == 0)
        def _():
            pltpu.make_async_copy(yl_hbm.at[idx_s.at[pl.ds(b * _BE, _BE)]],
                                  rows, sem).wait()

        @pl.when(c == 1)
        def _():
            pltpu.make_async_copy(yr_hbm.at[idx_s.at[pl.ds(b * _BE, _BE)]],
                                  rows, sem).wait()

    def scat(b, rows):
        pltpu.sync_copy(rows, acc_sh.at[idx_d.at[b]], add=True)

    start(0, rows_a, sem_a)

    def pair(g, _):
        b0 = 2 * g
        start(b0 + 1, rows_b, sem_b)
        wait(b0, rows_a, sem_a)
        scat(b0, rows_a)

        @pl.when(g < _NP - 1)
        def _():
            start(b0 + 2, rows_a, sem_a)

        wait(b0 + 1, rows_b, sem_b)
        scat(b0 + 1, rows_b)
        return 0

    lax.fori_loop(0, _NP, pair, 0)

    @pl.when(c == 0)
    def _():
        pltpu.async_copy(yl_hbm.at[idx_s.at[pl.ds(_NB * _BE, _TL)]],
                         rows_b.at[pl.ds(0, _TL)], sem_b)
        pltpu.make_async_copy(yl_hbm.at[idx_s.at[pl.ds(_NB * _BE, _TL)]],
                              rows_b.at[pl.ds(0, _TL)], sem_b).wait()

    @pl.when(c == 1)
    def _():
        pltpu.async_copy(yr_hbm.at[idx_s.at[pl.ds(_NB * _BE, _TL)]],
                         rows_b.at[pl.ds(0, _TL)], sem_b)
        pltpu.make_async_copy(yr_hbm.at[idx_s.at[pl.ds(_NB * _BE, _TL)]],
                              rows_b.at[pl.ds(0, _TL)], sem_b).wait()

    pltpu.sync_copy(rows_b.at[pl.ds(0, _TL)], acc_sh.at[idx_t], add=True)
    plsc.subcore_barrier()

    @pl.when((c == 0) & (s < 15))
    def _():
        pltpu.sync_copy(acc_sh.at[pl.ds(r0, _RT)], accl_hbm.at[pl.ds(r0, _RT)])

    @pl.when((c == 0) & (s == 15))
    def _():
        pltpu.sync_copy(acc_sh.at[pl.ds(15 * _RT, 640)],
                        accl_hbm.at[pl.ds(15 * _RT, 640)])

    @pl.when((c == 1) & (s < 15))
    def _():
        pltpu.sync_copy(acc_sh.at[pl.ds(r0, _RT)], accr_hbm.at[pl.ds(r0, _RT)])

    @pl.when((c == 1) & (s == 15))
    def _():
        pltpu.sync_copy(acc_sh.at[pl.ds(15 * _RT, 640)],
                        accr_hbm.at[pl.ds(15 * _RT, 640)])


_edge_call = pl.kernel(
    _edge_body,
    out_type=[
        jax.ShapeDtypeStruct((N, H), jnp.float32),
        jax.ShapeDtypeStruct((N, H), jnp.float32),
    ],
    mesh=plsc.VectorSubcoreMesh(core_axis_name="c", subcore_axis_name="s"),
    scratch_types=[
        pltpu.VMEM((_ET,), jnp.int32),
        pltpu.VMEM((_NB, _BE), jnp.int32),
        pltpu.VMEM((_TL,), jnp.int32),
        pltpu.VMEM((_BE, H), jnp.float32),
        pltpu.VMEM((_BE, H), jnp.float32),
        pltpu.VMEM_SHARED((N, H), jnp.float32),
        pltpu.SemaphoreType.DMA,
        pltpu.SemaphoreType.DMA,
        pltpu.SemaphoreType.DMA,
    ],
)


def _fin_body(al_ref, ar_ref, yl_ref, yr_ref, dis_ref, b_ref, out_ref):
    d = dis_ref[...]
    left = (al_ref[...] + yl_ref[...]) * d
    right = (ar_ref[...] + yr_ref[...]) * d
    out_ref[...] = jnp.concatenate([left, right], axis=1) + b_ref[...]


_fin_call = pl.pallas_call(
    _fin_body,
    grid=(N // _RB,),
    in_specs=[
        pl.BlockSpec((_RB, H), lambda i: (i, 0)),
        pl.BlockSpec((_RB, H), lambda i: (i, 0)),
        pl.BlockSpec((_RB, H), lambda i: (i, 0)),
        pl.BlockSpec((_RB, H), lambda i: (i, 0)),
        pl.BlockSpec((_RB, 1), lambda i: (i, 0)),
        pl.BlockSpec((1, D), lambda i: (0, 0)),
    ],
    out_specs=pl.BlockSpec((_RB, D), lambda i: (i, 0)),
    out_shape=jax.ShapeDtypeStruct((N, D), jnp.float32),
)


def kernel(x, edge_index, W, b):
    assert x.shape == (N, D) and W.shape == (D, D) and edge_index.shape == (2, E)
    src = edge_index[0].astype(jnp.int32)
    dst = edge_index[1].astype(jnp.int32)
    degs0, degs1 = _deg_call(dst)
    d0 = degs0[:N].reshape(N, 1)
    d1 = degs1[:N].reshape(N, 1)
    yl, yr, dis = _mm_call(x, W, d0, d1)
    accl, accr = _edge_call(yl, yr, src, dst)
    out = _fin_call(accl, accr, yl, yr, dis, b.reshape(1, D))
    return out
